# Initial kernel scaffold; baseline (speedup 1.0000x reference)
import jax
import jax.numpy as jnp
from jax.experimental import pallas as pl


def kernel(x, batch):
    raise NotImplementedError("write your pallas kernel here")


# speedup vs baseline: 4.3355x; 4.3355x over previous
import jax
import jax.numpy as jnp
from jax.experimental import pallas as pl

N = 50000
B = 128
PAD = 48
ROWS_2D = (N + PAD) // 128
BLK = 1000


def _bounds_kernel(bfull_ref, starts_ref, ends_ref, inv_ref):
    A = bfull_ref[...]
    lane = jax.lax.broadcasted_iota(jnp.int32, (1, B), 1)

    def body(b, carry):
        starts, ends = carry
        c = jnp.sum((A <= b).astype(jnp.int32))
        ends = ends + c * (lane == b).astype(jnp.int32)
        starts = starts + c * (lane == (b + 1)).astype(jnp.int32)
        return starts, ends

    z = jnp.zeros((1, B), jnp.int32)
    starts, ends = jax.lax.fori_loop(0, B, body, (z, z))
    ends = jnp.where(lane == (B - 1), N, ends)
    deg = (ends - starts).astype(jnp.float32)
    inv_ref[...] = jax.lax.rsqrt(jnp.maximum(deg, 1.0))
    starts_ref[...] = starts
    ends_ref[...] = ends


def _scale_kernel(x_ref, starts_ref, ends_ref, inv_ref, out_ref):
    i = pl.program_id(0)
    rows = jax.lax.broadcasted_iota(jnp.int32, (BLK, B), 0) + i * BLK
    oh = ((rows >= starts_ref[...]) & (rows < ends_ref[...])).astype(
        jnp.float32
    )
    scale = jax.lax.dot_general(
        oh, inv_ref[...], (((1,), (1,)), ((), ())),
        preferred_element_type=jnp.float32,
    )
    out_ref[...] = x_ref[...] * scale


def kernel(x, batch):
    b32 = batch.astype(jnp.int32)
    bfull = jnp.concatenate(
        [b32, jnp.full((PAD,), B - 1, jnp.int32)]
    ).reshape(ROWS_2D, 128)

    starts, ends, inv = pl.pallas_call(
        _bounds_kernel,
        out_shape=[
            jax.ShapeDtypeStruct((1, B), jnp.int32),
            jax.ShapeDtypeStruct((1, B), jnp.int32),
            jax.ShapeDtypeStruct((1, B), jnp.float32),
        ],
    )(bfull)

    D = x.shape[1]
    out = pl.pallas_call(
        _scale_kernel,
        grid=(N // BLK,),
        in_specs=[
            pl.BlockSpec((BLK, D), lambda i: (i, 0)),
            pl.BlockSpec((1, B), lambda i: (0, 0)),
            pl.BlockSpec((1, B), lambda i: (0, 0)),
            pl.BlockSpec((1, B), lambda i: (0, 0)),
        ],
        out_specs=pl.BlockSpec((BLK, D), lambda i: (i, 0)),
        out_shape=jax.ShapeDtypeStruct(x.shape, x.dtype),
    )(x, starts, ends, inv)
    return out


# speedup vs baseline: 5.5145x; 1.2719x over previous
import jax
import jax.numpy as jnp
from jax.experimental import pallas as pl

N = 50000
B = 128
PAD = 48
ROWS_2D = (N + PAD) // 128
BLK = 2000


def _bounds_kernel(bfull_ref, starts_ref, ends_ref, inv_ref):
    A = bfull_ref[...]
    lane = jax.lax.broadcasted_iota(jnp.int32, (1, B), 1)

    def body(b, carry):
        starts, ends = carry
        c = jnp.sum((A <= b).astype(jnp.int32))
        ends = ends + c * (lane == b).astype(jnp.int32)
        starts = starts + c * (lane == (b + 1)).astype(jnp.int32)
        return starts, ends

    z = jnp.zeros((1, B), jnp.int32)
    starts, ends = jax.lax.fori_loop(0, B, body, (z, z))
    ends = jnp.where(lane == (B - 1), N, ends)
    deg = (ends - starts).astype(jnp.float32)
    inv_ref[...] = jax.lax.rsqrt(jnp.maximum(deg, 1.0))
    starts_ref[...] = starts
    ends_ref[...] = ends


def _scale_kernel(x_ref, starts_ref, ends_ref, inv_ref, out_ref):
    i = pl.program_id(0)
    rows = jax.lax.broadcasted_iota(jnp.int32, (BLK, B), 0) + i * BLK
    oh = ((rows >= starts_ref[...]) & (rows < ends_ref[...])).astype(
        jnp.float32
    )
    scale = jax.lax.dot_general(
        oh, inv_ref[...], (((1,), (1,)), ((), ())),
        preferred_element_type=jnp.float32,
    )
    out_ref[...] = x_ref[...] * scale


def kernel(x, batch):
    b32 = batch.astype(jnp.int32)
    bfull = jnp.concatenate(
        [b32, jnp.full((PAD,), B - 1, jnp.int32)]
    ).reshape(ROWS_2D, 128)

    starts, ends, inv = pl.pallas_call(
        _bounds_kernel,
        out_shape=[
            jax.ShapeDtypeStruct((1, B), jnp.int32),
            jax.ShapeDtypeStruct((1, B), jnp.int32),
            jax.ShapeDtypeStruct((1, B), jnp.float32),
        ],
    )(bfull)

    D = x.shape[1]
    out = pl.pallas_call(
        _scale_kernel,
        grid=(N // BLK,),
        in_specs=[
            pl.BlockSpec((BLK, D), lambda i: (i, 0)),
            pl.BlockSpec((1, B), lambda i: (0, 0)),
            pl.BlockSpec((1, B), lambda i: (0, 0)),
            pl.BlockSpec((1, B), lambda i: (0, 0)),
        ],
        out_specs=pl.BlockSpec((BLK, D), lambda i: (i, 0)),
        out_shape=jax.ShapeDtypeStruct(x.shape, x.dtype),
    )(x, starts, ends, inv)
    return out


# speedup vs baseline: 5.9680x; 1.0822x over previous
import jax
import jax.numpy as jnp
from jax.experimental import pallas as pl

N = 50000
B = 128
PAD = 48
ROWS_2D = (N + PAD) // 128
BLK = 5000


def _bounds_kernel(bfull_ref, starts_ref, ends_ref, inv_ref):
    A = bfull_ref[...]
    lane = jax.lax.broadcasted_iota(jnp.int32, (1, B), 1)

    def body(b, carry):
        starts, ends = carry
        c = jnp.sum((A <= b).astype(jnp.int32))
        ends = ends + c * (lane == b).astype(jnp.int32)
        starts = starts + c * (lane == (b + 1)).astype(jnp.int32)
        return starts, ends

    z = jnp.zeros((1, B), jnp.int32)
    starts, ends = jax.lax.fori_loop(0, B, body, (z, z))
    ends = jnp.where(lane == (B - 1), N, ends)
    deg = (ends - starts).astype(jnp.float32)
    inv_ref[...] = jax.lax.rsqrt(jnp.maximum(deg, 1.0))
    starts_ref[...] = starts
    ends_ref[...] = ends


def _scale_kernel(x_ref, starts_ref, ends_ref, inv_ref, out_ref):
    i = pl.program_id(0)
    rows = jax.lax.broadcasted_iota(jnp.int32, (BLK, B), 0) + i * BLK
    oh = ((rows >= starts_ref[...]) & (rows < ends_ref[...])).astype(
        jnp.float32
    )
    scale = jax.lax.dot_general(
        oh, inv_ref[...], (((1,), (1,)), ((), ())),
        preferred_element_type=jnp.float32,
    )
    out_ref[...] = x_ref[...] * scale


def kernel(x, batch):
    b32 = batch.astype(jnp.int32)
    bfull = jnp.concatenate(
        [b32, jnp.full((PAD,), B - 1, jnp.int32)]
    ).reshape(ROWS_2D, 128)

    starts, ends, inv = pl.pallas_call(
        _bounds_kernel,
        out_shape=[
            jax.ShapeDtypeStruct((1, B), jnp.int32),
            jax.ShapeDtypeStruct((1, B), jnp.int32),
            jax.ShapeDtypeStruct((1, B), jnp.float32),
        ],
    )(bfull)

    D = x.shape[1]
    out = pl.pallas_call(
        _scale_kernel,
        grid=(N // BLK,),
        in_specs=[
            pl.BlockSpec((BLK, D), lambda i: (i, 0)),
            pl.BlockSpec((1, B), lambda i: (0, 0)),
            pl.BlockSpec((1, B), lambda i: (0, 0)),
            pl.BlockSpec((1, B), lambda i: (0, 0)),
        ],
        out_specs=pl.BlockSpec((BLK, D), lambda i: (i, 0)),
        out_shape=jax.ShapeDtypeStruct(x.shape, x.dtype),
    )(x, starts, ends, inv)
    return out


# speedup vs baseline: 6.1316x; 1.0274x over previous
import jax
import jax.numpy as jnp
from jax.experimental import pallas as pl

N = 50000
B = 128
PAD = 48
ROWS_2D = (N + PAD) // 128
BLK = 10000


def _bounds_kernel(bfull_ref, starts_ref, ends_ref, inv_ref):
    A = bfull_ref[...]
    lane = jax.lax.broadcasted_iota(jnp.int32, (1, B), 1)

    def body(b, carry):
        starts, ends = carry
        c = jnp.sum((A <= b).astype(jnp.int32))
        ends = ends + c * (lane == b).astype(jnp.int32)
        starts = starts + c * (lane == (b + 1)).astype(jnp.int32)
        return starts, ends

    z = jnp.zeros((1, B), jnp.int32)
    starts, ends = jax.lax.fori_loop(0, B, body, (z, z))
    ends = jnp.where(lane == (B - 1), N, ends)
    deg = (ends - starts).astype(jnp.float32)
    inv_ref[...] = jax.lax.rsqrt(jnp.maximum(deg, 1.0))
    starts_ref[...] = starts
    ends_ref[...] = ends


def _scale_kernel(x_ref, starts_ref, ends_ref, inv_ref, out_ref):
    i = pl.program_id(0)
    rows = jax.lax.broadcasted_iota(jnp.int32, (BLK, B), 0) + i * BLK
    oh = ((rows >= starts_ref[...]) & (rows < ends_ref[...])).astype(
        jnp.float32
    )
    scale = jax.lax.dot_general(
        oh, inv_ref[...], (((1,), (1,)), ((), ())),
        preferred_element_type=jnp.float32,
    )
    out_ref[...] = x_ref[...] * scale


def kernel(x, batch):
    b32 = batch.astype(jnp.int32)
    bfull = jnp.concatenate(
        [b32, jnp.full((PAD,), B - 1, jnp.int32)]
    ).reshape(ROWS_2D, 128)

    starts, ends, inv = pl.pallas_call(
        _bounds_kernel,
        out_shape=[
            jax.ShapeDtypeStruct((1, B), jnp.int32),
            jax.ShapeDtypeStruct((1, B), jnp.int32),
            jax.ShapeDtypeStruct((1, B), jnp.float32),
        ],
    )(bfull)

    D = x.shape[1]
    out = pl.pallas_call(
        _scale_kernel,
        grid=(N // BLK,),
        in_specs=[
            pl.BlockSpec((BLK, D), lambda i: (i, 0)),
            pl.BlockSpec((1, B), lambda i: (0, 0)),
            pl.BlockSpec((1, B), lambda i: (0, 0)),
            pl.BlockSpec((1, B), lambda i: (0, 0)),
        ],
        out_specs=pl.BlockSpec((BLK, D), lambda i: (i, 0)),
        out_shape=jax.ShapeDtypeStruct(x.shape, x.dtype),
    )(x, starts, ends, inv)
    return out
